# Initial kernel scaffold; baseline (speedup 1.0000x reference)
import jax
import jax.numpy as jnp
from jax.experimental import pallas as pl


def kernel(x, edge_index, W1, b1, W2, b2):
    raise NotImplementedError("write your pallas kernel here")


# speedup vs baseline: 16.3929x; 16.3929x over previous
import functools

import jax
import jax.numpy as jnp
from jax import lax
from jax.experimental import pallas as pl
from jax.experimental.pallas import tpu as pltpu
from jax.experimental.pallas import tpu_sc as plsc

N_NODES = 10000
N_PAD = 10240
JUNK_ROW = 10016
E_RAW = 320000
TILES = 16
CHUNK = 128
NCH = 157
E_PAD = TILES * NCH * CHUNK
RPT = N_PAD // TILES
D_HALF = 128
BLK = 512

_mesh = plsc.VectorSubcoreMesh(
    core_axis_name="c", subcore_axis_name="s", num_cores=2, num_subcores=16
)


@functools.partial(
    pl.kernel,
    out_type=jax.ShapeDtypeStruct((N_PAD,), jnp.float32),
    mesh=_mesh,
    scratch_types=[
        pltpu.VMEM((NCH, CHUNK), jnp.int32),
        pltpu.VMEM((CHUNK,), jnp.float32),
        pltpu.VMEM_SHARED((N_PAD,), jnp.float32),
        pltpu.VMEM((RPT,), jnp.float32),
    ],
)
def _deg_kernel(dst3, deg_out, dst2d, ones_v, acc_sh, outb):
    c = lax.axis_index("c")
    s = lax.axis_index("s")

    @pl.when(c == 0)
    def _():
        pltpu.sync_copy(dst3.at[s], dst2d)
        for q in range(CHUNK // 16):
            ones_v[pl.ds(q * 16, 16)] = jnp.ones((16,), jnp.float32)

        def zb(i, _):
            outb[pl.ds(i * 16, 16)] = jnp.zeros((16,), jnp.float32)
            return 0

        lax.fori_loop(0, RPT // 16, zb, 0)
        pltpu.sync_copy(outb, acc_sh.at[pl.ds(s * RPT, RPT)])

    plsc.subcore_barrier()

    @pl.when(c == 0)
    def _():
        def body(j, _):
            pltpu.sync_copy(ones_v, acc_sh.at[dst2d.at[j]], add=True)
            return 0

        lax.fori_loop(0, NCH, body, 0)

    plsc.subcore_barrier()

    @pl.when(c == 0)
    def _():
        base = s * RPT
        pltpu.sync_copy(acc_sh.at[pl.ds(base, RPT)], outb)

        def fb(v, _):
            sl = pl.ds(v * 16, 16)
            outb[sl] = outb[sl] + 1.0
            return 0

        lax.fori_loop(0, RPT // 16, fb, 0)
        pltpu.sync_copy(outb, deg_out.at[pl.ds(base, RPT)])


def _mm1_body(x_ref, w_ref, b_ref, deg_ref, h1_ref, h2_ref, dinv_ref):
    dinv = lax.rsqrt(deg_ref[...])
    h = jnp.dot(x_ref[...], w_ref[...], preferred_element_type=jnp.float32)
    hp = (h + b_ref[...]) * dinv
    h1_ref[...] = hp[:, :D_HALF]
    h2_ref[...] = hp[:, D_HALF:]
    dinv_ref[...] = dinv


_mm1 = pl.pallas_call(
    _mm1_body,
    grid=(N_PAD // BLK,),
    in_specs=[
        pl.BlockSpec((BLK, 128), lambda i: (i, 0)),
        pl.BlockSpec((128, 256), lambda i: (0, 0)),
        pl.BlockSpec((1, 256), lambda i: (0, 0)),
        pl.BlockSpec((BLK, 1), lambda i: (i, 0)),
    ],
    out_specs=[
        pl.BlockSpec((BLK, D_HALF), lambda i: (i, 0)),
        pl.BlockSpec((BLK, D_HALF), lambda i: (i, 0)),
        pl.BlockSpec((BLK, 1), lambda i: (i, 0)),
    ],
    out_shape=[
        jax.ShapeDtypeStruct((N_PAD, D_HALF), jnp.float32),
        jax.ShapeDtypeStruct((N_PAD, D_HALF), jnp.float32),
        jax.ShapeDtypeStruct((N_PAD, 1), jnp.float32),
    ],
)


@functools.partial(
    pl.kernel,
    out_type=[
        jax.ShapeDtypeStruct((N_PAD, D_HALF), jnp.float32),
        jax.ShapeDtypeStruct((N_PAD, D_HALF), jnp.float32),
    ],
    mesh=_mesh,
    scratch_types=[
        pltpu.VMEM((2, CHUNK), jnp.int32),
        pltpu.VMEM((CHUNK, D_HALF), jnp.float32),
        pltpu.VMEM_SHARED((N_PAD, D_HALF), jnp.float32),
        pltpu.SemaphoreType.DMA,
    ],
)
def _agg_kernel(h1, h2, e4, s1_out, s2_out, idx2, rows, acc_sh, sem):
    c = lax.axis_index("c")
    s = lax.axis_index("s")
    base = s * RPT

    @pl.when(c == 0)
    def _():
        pltpu.sync_copy(h1.at[pl.ds(base, RPT)], acc_sh.at[pl.ds(base, RPT)])

    @pl.when(c == 1)
    def _():
        pltpu.sync_copy(h2.at[pl.ds(base, RPT)], acc_sh.at[pl.ds(base, RPT)])

    plsc.subcore_barrier()

    @pl.when(c == 0)
    def _():
        def body(j, _):
            pltpu.sync_copy(e4.at[s, j], idx2)
            pltpu.async_copy(h1.at[idx2.at[0]], rows, sem).wait()
            pltpu.sync_copy(rows, acc_sh.at[idx2.at[1]], add=True)
            return 0

        lax.fori_loop(0, NCH, body, 0)

    @pl.when(c == 1)
    def _():
        def body(j, _):
            pltpu.sync_copy(e4.at[s, j], idx2)
            pltpu.async_copy(h2.at[idx2.at[0]], rows, sem).wait()
            pltpu.sync_copy(rows, acc_sh.at[idx2.at[1]], add=True)
            return 0

        lax.fori_loop(0, NCH, body, 0)

    plsc.subcore_barrier()

    @pl.when(c == 0)
    def _():
        pltpu.sync_copy(acc_sh.at[pl.ds(base, RPT)], s1_out.at[pl.ds(base, RPT)])

    @pl.when(c == 1)
    def _():
        pltpu.sync_copy(acc_sh.at[pl.ds(base, RPT)], s2_out.at[pl.ds(base, RPT)])


def _mm2_body(s1_ref, s2_ref, dinv_ref, w2a_ref, w2b_ref, b2_ref, g_ref):
    dinv = dinv_ref[...]
    ra = jnp.maximum(dinv * s1_ref[...], 0.0)
    rb = jnp.maximum(dinv * s2_ref[...], 0.0)
    t = (jnp.dot(ra, w2a_ref[...], preferred_element_type=jnp.float32)
         + jnp.dot(rb, w2b_ref[...], preferred_element_type=jnp.float32))
    g_ref[...] = dinv * (t + b2_ref[...])


_mm2 = pl.pallas_call(
    _mm2_body,
    grid=(N_PAD // BLK,),
    in_specs=[
        pl.BlockSpec((BLK, D_HALF), lambda i: (i, 0)),
        pl.BlockSpec((BLK, D_HALF), lambda i: (i, 0)),
        pl.BlockSpec((BLK, 1), lambda i: (i, 0)),
        pl.BlockSpec((D_HALF, 1), lambda i: (0, 0)),
        pl.BlockSpec((D_HALF, 1), lambda i: (0, 0)),
        pl.BlockSpec((1, 1), lambda i: (0, 0)),
    ],
    out_specs=pl.BlockSpec((BLK, 1), lambda i: (i, 0)),
    out_shape=jax.ShapeDtypeStruct((N_PAD, 1), jnp.float32),
)


@functools.partial(
    pl.kernel,
    out_type=jax.ShapeDtypeStruct((N_PAD,), jnp.float32),
    mesh=_mesh,
    scratch_types=[
        pltpu.VMEM((NCH, CHUNK), jnp.int32),
        pltpu.VMEM((NCH, CHUNK), jnp.int32),
        pltpu.VMEM((CHUNK,), jnp.float32),
        pltpu.VMEM_SHARED((N_PAD,), jnp.float32),
        pltpu.VMEM((RPT,), jnp.float32),
        pltpu.VMEM((RPT,), jnp.float32),
        pltpu.VMEM((RPT,), jnp.float32),
        pltpu.SemaphoreType.DMA,
    ],
)
def _l2_kernel(src3, dst3, g_hbm, dinv_hbm, out_hbm,
               src2d, dst2d, vals, acc_sh, outb, gb, db, sem):
    c = lax.axis_index("c")
    s = lax.axis_index("s")

    @pl.when(c == 0)
    def _():
        pltpu.sync_copy(src3.at[s], src2d)
        pltpu.sync_copy(dst3.at[s], dst2d)

        def zb(i, _):
            outb[pl.ds(i * 16, 16)] = jnp.zeros((16,), jnp.float32)
            return 0

        lax.fori_loop(0, RPT // 16, zb, 0)
        pltpu.sync_copy(outb, acc_sh.at[pl.ds(s * RPT, RPT)])

    plsc.subcore_barrier()

    @pl.when(c == 0)
    def _():
        def body(j, _):
            pltpu.async_copy(g_hbm.at[src2d.at[j]], vals, sem).wait()
            pltpu.sync_copy(vals, acc_sh.at[dst2d.at[j]], add=True)
            return 0

        lax.fori_loop(0, NCH, body, 0)

    plsc.subcore_barrier()

    @pl.when(c == 0)
    def _():
        base = s * RPT
        pltpu.sync_copy(acc_sh.at[pl.ds(base, RPT)], outb)
        pltpu.sync_copy(g_hbm.at[pl.ds(base, RPT)], gb)
        pltpu.sync_copy(dinv_hbm.at[pl.ds(base, RPT)], db)

        def fb(v, _):
            sl = pl.ds(v * 16, 16)
            outb[sl] = (outb[sl] + gb[sl]) * db[sl]
            return 0

        lax.fori_loop(0, RPT // 16, fb, 0)
        pltpu.sync_copy(outb, out_hbm.at[pl.ds(base, RPT)])


def kernel(x, edge_index, W1, b1, W2, b2):
    src = edge_index[0]
    dst = edge_index[1]
    pad = E_PAD - E_RAW
    srcp = jnp.concatenate([src, jnp.zeros((pad,), src.dtype)])
    dstp = jnp.concatenate([dst, jnp.full((pad,), JUNK_ROW, dst.dtype)])
    src3 = srcp.reshape(TILES, NCH, CHUNK)
    dst3 = dstp.reshape(TILES, NCH, CHUNK)
    e4 = jnp.stack([src3, dst3], axis=2)
    x_pad = jnp.zeros((N_PAD, x.shape[1]), x.dtype).at[:N_NODES].set(x)

    deg = _deg_kernel(dst3)
    h1, h2, dinv2 = _mm1(x_pad, W1, b1.reshape(1, -1), deg.reshape(N_PAD, 1))
    s1, s2 = _agg_kernel(h1, h2, e4)
    g2 = _mm2(s1, s2, dinv2, W2[:D_HALF], W2[D_HALF:], b2.reshape(1, 1))
    outp = _l2_kernel(src3, dst3, g2.reshape(N_PAD), dinv2.reshape(N_PAD))
    return outp[:N_NODES].reshape(N_NODES, 1)
